# baseline (speedup 1.0000x reference)
import functools

import jax
import jax.numpy as jnp
from jax import lax
from jax.experimental import pallas as pl
from jax.experimental.pallas import tpu as pltpu
from jax.experimental.pallas import tpu_sc as plsc

_NUM_CORES = 2
_NUM_SUBCORES = 16
_NW = _NUM_CORES * _NUM_SUBCORES
_LANES = 16

_CHUNK = 128


def _proj_body(x_ref, w_ref, b_ref, o_ref):
    o_ref[...] = (
        jnp.dot(x_ref[...], w_ref[...], preferred_element_type=jnp.float32)
        + b_ref[...]
    )


def _precompute_tables(embed_table, proj_w, proj_b, row_tile):
    v, d = embed_table.shape
    s = proj_w.shape[1] // d
    half = s // 2
    wt = proj_w.reshape(d, s, d).transpose(1, 2, 0)
    w2 = jnp.concatenate([wt[:half], wt[half:]], axis=2).reshape(half * d, 2 * d)
    bias = jnp.tile((proj_b / s).astype(jnp.float32), 2).reshape(1, 2 * d)
    n_row_tiles = v // row_tile
    return pl.pallas_call(
        _proj_body,
        grid=(n_row_tiles, half),
        in_specs=[
            pl.BlockSpec((row_tile, d), lambda k, a: (k, 0)),
            pl.BlockSpec((d, 2 * d), lambda k, a: (a, 0)),
            pl.BlockSpec((1, 2 * d), lambda k, a: (0, 0)),
        ],
        out_specs=pl.BlockSpec(
            (row_tile, 2 * d), lambda k, a: (a * n_row_tiles + k, 0)
        ),
        out_shape=jax.ShapeDtypeStruct((half * v, 2 * d), jnp.float32),
    )(embed_table, w2, bias)


def _gather_sum(flat_idx, tables, b_dim, t_dim, s, d):
    n_tokens = b_dim * t_dim
    per_w = n_tokens // _NW
    rows_per_w = per_w // t_dim
    c0 = (t_dim // 2 + 7) // 8 * 8
    chunk_len = (c0, t_dim - c0)
    chunk_off = (0, c0)
    n_chunks = 2 * rows_per_w

    mesh = plsc.VectorSubcoreMesh(
        core_axis_name="c",
        subcore_axis_name="s",
        num_cores=_NUM_CORES,
        num_subcores=_NUM_SUBCORES,
    )

    @functools.partial(
        pl.kernel,
        mesh=mesh,
        compiler_params=pltpu.CompilerParams(use_tc_tiling_on_sc=False),
        out_type=jax.ShapeDtypeStruct((b_dim, t_dim, d), jnp.float32),
        scratch_types=[
            pltpu.VMEM((s, per_w), jnp.int32),
            pltpu.VMEM((2, s, c0, d), jnp.float32),
            pltpu.VMEM((2, c0, d), jnp.float32),
            pltpu.SemaphoreType.DMA((2,)),
            pltpu.SemaphoreType.DMA((2,)),
        ],
    )
    def sc_kernel(idx_hbm, tab_hbm, out_hbm, idx_all, rows_v, out_v, gsem, osem):
        wid = lax.axis_index("s") * _NUM_CORES + lax.axis_index("c")
        base0 = wid * per_w
        row0 = wid * rows_per_w
        pltpu.sync_copy(idx_hbm.at[:, pl.ds(base0, per_w)], idx_all)

        def gather_copy(r, half, j):
            c = chunk_len[half]
            return pltpu.make_async_copy(
                tab_hbm.at[idx_all.at[j, pl.ds(r * t_dim + chunk_off[half], c)]],
                rows_v.at[half, j, pl.ds(0, c)],
                gsem.at[half],
            )

        def out_copy(r, half):
            c = chunk_len[half]
            return pltpu.make_async_copy(
                out_v.at[half, pl.ds(0, c)],
                out_hbm.at[row0 + r, pl.ds(chunk_off[half], c), :],
                osem.at[half],
            )

        def fire(r, half):
            for j in range(s):
                gather_copy(r, half, j).start()

        def compute(half):
            def tok_body(t, carry):
                for k in range(d // _LANES):
                    col = pl.ds(k * _LANES, _LANES)
                    acc = rows_v[half, 0, t, col] + rows_v[half, 1, t, col]
                    for j in range(2, s):
                        acc = acc + rows_v[half, j, t, col]
                    out_v[half, t, col] = acc
                return carry

            lax.fori_loop(0, chunk_len[half], tok_body, 0, unroll=4)

        fire(0, 0)

        def outer(r, carry):
            for half in range(2):
                ch = 2 * r + half
                if half == 0:
                    nr, nh = r, 1
                else:
                    nr, nh = r + 1, 0

                @pl.when(ch + 1 < n_chunks)
                def _():
                    fire(nr, nh)

                for j in range(s):
                    gather_copy(r, half, j).wait()

                @pl.when(ch >= 2)
                def _():
                    out_copy(r - 1, half).wait()

                compute(half)
                out_copy(r, half).start()
            return carry

        lax.fori_loop(0, rows_per_w, outer, 0)
        out_copy(rows_per_w - 1, 0).wait()
        out_copy(rows_per_w - 1, 1).wait()

    return sc_kernel(flat_idx, tables)


def kernel(ctrl_tokens, embed_table, proj_w, proj_b):
    b, t, s = ctrl_tokens.shape
    v, d = embed_table.shape
    n = b * t

    tab2 = _precompute_tables(embed_table, proj_w, proj_b, row_tile=10000)
    tables = tab2.reshape(s * v, d)

    idx = ctrl_tokens.reshape(n, s).astype(jnp.int32)
    j = jnp.arange(s, dtype=jnp.int32)
    flat_idx = (2 * ((j % 2)[None, :] * v + idx) + (j // 2)[None, :]).T

    return _gather_sum(flat_idx, tables, b, t, s, d)


# speedup vs baseline: 1.0229x; 1.0229x over previous
import functools

import jax
import jax.numpy as jnp
from jax import lax
from jax.experimental import pallas as pl
from jax.experimental.pallas import tpu as pltpu
from jax.experimental.pallas import tpu_sc as plsc

_NUM_CORES = 2
_NUM_SUBCORES = 16
_NW = _NUM_CORES * _NUM_SUBCORES
_LANES = 16

_CHUNK = 128


def _proj_body(x_ref, w_ref, b_ref, o_ref):
    o_ref[...] = (
        jnp.dot(x_ref[...], w_ref[...], preferred_element_type=jnp.float32)
        + b_ref[...]
    )


def _precompute_tables(embed_table, proj_w, proj_b, row_tile):
    v, d = embed_table.shape
    s = proj_w.shape[1] // d
    half = s // 2
    wt = proj_w.reshape(d, s, d).transpose(1, 2, 0)
    w2 = jnp.concatenate([wt[:half], wt[half:]], axis=2).reshape(half * d, 2 * d)
    bias = jnp.tile((proj_b / s).astype(jnp.float32), 2).reshape(1, 2 * d)
    n_row_tiles = v // row_tile
    return pl.pallas_call(
        _proj_body,
        grid=(n_row_tiles, half),
        in_specs=[
            pl.BlockSpec((row_tile, d), lambda k, a: (k, 0)),
            pl.BlockSpec((d, 2 * d), lambda k, a: (a, 0)),
            pl.BlockSpec((1, 2 * d), lambda k, a: (0, 0)),
        ],
        out_specs=pl.BlockSpec(
            (row_tile, 2 * d), lambda k, a: (a * n_row_tiles + k, 0)
        ),
        out_shape=jax.ShapeDtypeStruct((half * v, 2 * d), jnp.float32),
    )(embed_table, w2, bias)


def _gather_sum(flat_idx, tables, n_tokens, s, d):
    per_w = n_tokens // _NW
    n_chunks = per_w // _CHUNK
    assert n_chunks % 2 == 0

    mesh = plsc.VectorSubcoreMesh(
        core_axis_name="c",
        subcore_axis_name="s",
        num_cores=_NUM_CORES,
        num_subcores=_NUM_SUBCORES,
    )

    @functools.partial(
        pl.kernel,
        mesh=mesh,
        compiler_params=pltpu.CompilerParams(use_tc_tiling_on_sc=False),
        out_type=jax.ShapeDtypeStruct((n_tokens // 2, 2 * d), jnp.float32),
        scratch_types=[
            pltpu.VMEM((s, per_w), jnp.int32),
            pltpu.VMEM((2, s, _CHUNK, d), jnp.float32),
            pltpu.VMEM((2, _CHUNK // 2, 2 * d), jnp.float32),
            pltpu.SemaphoreType.DMA((2,)),
            pltpu.SemaphoreType.DMA((2,)),
        ],
    )
    def sc_kernel(idx_hbm, tab_hbm, out_hbm, idx_all, rows_v, out_v, gsem, osem):
        wid = lax.axis_index("s") * _NUM_CORES + lax.axis_index("c")
        base0 = wid * per_w
        pltpu.sync_copy(idx_hbm.at[:, pl.ds(base0, per_w)], idx_all)

        def gather_copy(buf, g, j):
            return pltpu.make_async_copy(
                tab_hbm.at[idx_all.at[j, pl.ds(g * _CHUNK, _CHUNK)]],
                rows_v.at[buf, j],
                gsem.at[buf],
            )

        def out_copy(buf, g):
            return pltpu.make_async_copy(
                out_v.at[buf],
                out_hbm.at[pl.ds((base0 + g * _CHUNK) // 2, _CHUNK // 2), :],
                osem.at[buf],
            )

        def fire(buf, g):
            for j in range(s):
                gather_copy(buf, g, j).start()

        def compute(buf):
            def pair_body(p, carry):
                for parity in range(2):
                    t = 2 * p + parity
                    for k in range(d // _LANES):
                        col = pl.ds(k * _LANES, _LANES)
                        ocol = pl.ds(parity * d + k * _LANES, _LANES)
                        acc = rows_v[buf, 0, t, col] + rows_v[buf, 1, t, col]
                        for j in range(2, s):
                            acc = acc + rows_v[buf, j, t, col]
                        out_v[buf, p, ocol] = acc
                return carry

            lax.fori_loop(0, _CHUNK // 2, pair_body, 0, unroll=2)

        fire(0, 0)

        def outer(i, carry):
            for buf in range(2):
                g = i * 2 + buf

                @pl.when(g + 1 < n_chunks)
                def _():
                    fire(1 - buf, g + 1)

                for j in range(s):
                    gather_copy(buf, g, j).wait()

                @pl.when(g >= 2)
                def _():
                    out_copy(buf, g - 2).wait()

                compute(buf)
                out_copy(buf, g).start()
            return carry

        lax.fori_loop(0, n_chunks // 2, outer, 0)
        out_copy(0, n_chunks - 2).wait()
        out_copy(1, n_chunks - 1).wait()

    return sc_kernel(flat_idx, tables)


def kernel(ctrl_tokens, embed_table, proj_w, proj_b):
    b, t, s = ctrl_tokens.shape
    v, d = embed_table.shape
    n = b * t

    tab2 = _precompute_tables(embed_table, proj_w, proj_b, row_tile=10000)
    tables = tab2.reshape(s * v, d)

    idx = ctrl_tokens.reshape(n, s).astype(jnp.int32)
    j = jnp.arange(s, dtype=jnp.int32)
    flat_idx = (2 * ((j % 2)[None, :] * v + idx) + (j // 2)[None, :]).T

    packed = _gather_sum(flat_idx, tables, n, s, d)
    return packed.reshape(b, t, d)


# speedup vs baseline: 1.1824x; 1.1560x over previous
import functools

import jax
import jax.numpy as jnp
from jax import lax
from jax.experimental import pallas as pl
from jax.experimental.pallas import tpu as pltpu
from jax.experimental.pallas import tpu_sc as plsc

_NUM_CORES = 2
_NUM_SUBCORES = 16
_NW = _NUM_CORES * _NUM_SUBCORES
_LANES = 16

_CHUNK = 128


def _proj_body(xt_ref, w_ref, b_ref, o_ref):
    o_ref[...] = (
        jax.lax.dot_general(
            xt_ref[...],
            w_ref[...],
            dimension_numbers=(((0,), (0,)), ((), ())),
            preferred_element_type=jnp.float32,
        )
        + b_ref[...]
    )


def _precompute_tables(embed_table, proj_w, proj_b, row_tile, v_pad):
    v, d = embed_table.shape
    s = proj_w.shape[1] // d
    half = s // 2
    wt = proj_w.reshape(d, s, d).transpose(1, 2, 0)
    w2 = jnp.concatenate([wt[:half], wt[half:]], axis=2).reshape(half * d, 2 * d)
    bias = jnp.tile((proj_b / s).astype(jnp.float32), 2).reshape(1, 2 * d)
    n_row_tiles = v_pad // row_tile
    return pl.pallas_call(
        _proj_body,
        grid=(n_row_tiles, half),
        in_specs=[
            pl.BlockSpec((d, row_tile), lambda k, a: (0, k)),
            pl.BlockSpec((d, 2 * d), lambda k, a: (a, 0)),
            pl.BlockSpec((1, 2 * d), lambda k, a: (0, 0)),
        ],
        out_specs=pl.BlockSpec(
            (row_tile, 2 * d), lambda k, a: (a * n_row_tiles + k, 0)
        ),
        out_shape=jax.ShapeDtypeStruct((half * v_pad, 2 * d), jnp.float32),
    )(embed_table.T, w2, bias)


def _gather_sum(flat_idx, tables, n_tokens, s, d):
    per_w = n_tokens // _NW
    n_chunks = per_w // _CHUNK
    assert n_chunks % 2 == 0

    mesh = plsc.VectorSubcoreMesh(
        core_axis_name="c",
        subcore_axis_name="s",
        num_cores=_NUM_CORES,
        num_subcores=_NUM_SUBCORES,
    )

    @functools.partial(
        pl.kernel,
        mesh=mesh,
        compiler_params=pltpu.CompilerParams(use_tc_tiling_on_sc=False),
        out_type=jax.ShapeDtypeStruct((n_tokens // 2, 2 * d), jnp.float32),
        scratch_types=[
            pltpu.VMEM((s, per_w), jnp.int32),
            pltpu.VMEM((2, s, _CHUNK, d), jnp.float32),
            pltpu.VMEM((2, _CHUNK // 2, 2 * d), jnp.float32),
            pltpu.SemaphoreType.DMA((2,)),
            pltpu.SemaphoreType.DMA((2,)),
        ],
    )
    def sc_kernel(idx_hbm, tab_hbm, out_hbm, idx_all, rows_v, out_v, gsem, osem):
        wid = lax.axis_index("s") * _NUM_CORES + lax.axis_index("c")
        base0 = wid * per_w
        pltpu.sync_copy(idx_hbm.at[:, pl.ds(base0, per_w)], idx_all)

        def gather_copy(buf, g, j):
            return pltpu.make_async_copy(
                tab_hbm.at[idx_all.at[j, pl.ds(g * _CHUNK, _CHUNK)]],
                rows_v.at[buf, j],
                gsem.at[buf],
            )

        def out_copy(buf, g):
            return pltpu.make_async_copy(
                out_v.at[buf],
                out_hbm.at[pl.ds((base0 + g * _CHUNK) // 2, _CHUNK // 2), :],
                osem.at[buf],
            )

        def fire(buf, g):
            for j in range(s):
                gather_copy(buf, g, j).start()

        def compute(buf):
            def pair_body(p, carry):
                for parity in range(2):
                    t = 2 * p + parity
                    for k in range(d // _LANES):
                        col = pl.ds(k * _LANES, _LANES)
                        ocol = pl.ds(parity * d + k * _LANES, _LANES)
                        acc = rows_v[buf, 0, t, col] + rows_v[buf, 1, t, col]
                        for j in range(2, s):
                            acc = acc + rows_v[buf, j, t, col]
                        out_v[buf, p, ocol] = acc
                return carry

            lax.fori_loop(0, _CHUNK // 2, pair_body, 0, unroll=2)

        fire(0, 0)

        def outer(i, carry):
            for buf in range(2):
                g = i * 2 + buf

                @pl.when(g + 1 < n_chunks)
                def _():
                    fire(1 - buf, g + 1)

                for j in range(s):
                    gather_copy(buf, g, j).wait()

                @pl.when(g >= 2)
                def _():
                    out_copy(buf, g - 2).wait()

                compute(buf)
                out_copy(buf, g).start()
            return carry

        lax.fori_loop(0, n_chunks // 2, outer, 0)
        out_copy(0, n_chunks - 2).wait()
        out_copy(1, n_chunks - 1).wait()

    return sc_kernel(flat_idx, tables)


def kernel(ctrl_tokens, embed_table, proj_w, proj_b):
    b, t, s = ctrl_tokens.shape
    v, d = embed_table.shape
    n = b * t

    v_pad = 12800 * ((v + 12799) // 12800)
    tab2 = _precompute_tables(embed_table, proj_w, proj_b, 12800, v_pad)
    tables = tab2.reshape(s * v_pad, d)

    idx = ctrl_tokens.reshape(n, s).astype(jnp.int32)
    j = jnp.arange(s, dtype=jnp.int32)
    flat_idx = (2 * ((j % 2)[None, :] * v_pad + idx) + (j // 2)[None, :]).T

    packed = _gather_sum(flat_idx, tables, n, s, d)
    return packed.reshape(b, t, d)
